# baseline (speedup 1.0000x reference)
import functools

import jax
import jax.numpy as jnp
from jax import lax
from jax.experimental import pallas as pl
from jax.experimental.pallas import tpu as pltpu
from jax.experimental.pallas import tpu_sc as plsc

N = 10000
E = 320000
D = 128
C = 128
G = 64

NC, NS, L = 2, 16, 16
NW = NC * NS
N_PAD = 10240
NPT = N_PAD // NS
K = 64
NIT0 = 132
NIT1 = 192
NITP = NIT0 + NIT1
E_PAD = NS * NITP * K
BR = 512
NBLK = N_PAD // BR


def _tc_pre_body(x_ref, wi_ref, bi_ref, wc_ref, as_ref, ad_ref,
                 xl_ref, asrc_ref, adst_ref):
    h = jnp.dot(x_ref[...], wi_ref[...], preferred_element_type=jnp.float32)
    h = h + bi_ref[...]
    xl = jnp.dot(h, wc_ref[...], preferred_element_type=jnp.float32)
    xl_ref[...] = xl
    asrc_ref[...] = jnp.sum(xl * as_ref[...], axis=1, keepdims=True)
    adst_ref[...] = jnp.sum(xl * ad_ref[...], axis=1, keepdims=True)


def _tc_pre(x_pad, W_in, b_in, W_conv, att_s, att_d):
    return pl.pallas_call(
        _tc_pre_body,
        grid=(NBLK,),
        in_specs=[
            pl.BlockSpec((BR, D), lambda i: (i, 0)),
            pl.BlockSpec((D, C), lambda i: (0, 0)),
            pl.BlockSpec((1, C), lambda i: (0, 0)),
            pl.BlockSpec((C, C), lambda i: (0, 0)),
            pl.BlockSpec((1, C), lambda i: (0, 0)),
            pl.BlockSpec((1, C), lambda i: (0, 0)),
        ],
        out_specs=[
            pl.BlockSpec((BR, C), lambda i: (i, 0)),
            pl.BlockSpec((BR, 1), lambda i: (i, 0)),
            pl.BlockSpec((BR, 1), lambda i: (i, 0)),
        ],
        out_shape=[
            jax.ShapeDtypeStruct((N_PAD, C), jnp.float32),
            jax.ShapeDtypeStruct((N_PAD, 1), jnp.float32),
            jax.ShapeDtypeStruct((N_PAD, 1), jnp.float32),
        ],
    )(x_pad, W_in, b_in, W_conv, att_s, att_d)


def _sc_body(ei_hbm, xl_hbm, asrc_hbm, adst_hbm,
             acc_out, den_out,
             asrc_v, adst_v, den_v, idx_buf, row_buf, w_buf,
             acc_sh, sem_i0, sem_i1, sem_i2, sem_r0, sem_r1, sem_s0, sem_s1):
    c = lax.axis_index("c")
    s = lax.axis_index("s")
    g = c * NS + s
    sems_i = (sem_i0, sem_i1, sem_i2)
    sems_r = (sem_r0, sem_r1)
    sems_s = (sem_s0, sem_s1)

    base = s * NITP + jnp.where(c == 0, 0, NIT0)
    nit = jnp.where(c == 0, NIT0, NIT1)
    ngrp = jnp.where(c == 0, NIT0 // 6, NIT1 // 6)

    def issue_idx(j, slot):
        om = jnp.minimum((base + j) * K, E - K)
        pltpu.async_copy(ei_hbm.at[0].at[pl.ds(om, K)], idx_buf.at[slot, 0],
                         sems_i[slot])
        pltpu.async_copy(ei_hbm.at[1].at[pl.ds(om, K)], idx_buf.at[slot, 1],
                         sems_i[slot])

    def wait_idx(j, slot):
        om = jnp.minimum((base + j) * K, E - K)
        pltpu.make_async_copy(ei_hbm.at[0].at[pl.ds(om, K)],
                              idx_buf.at[slot, 0], sems_i[slot]).wait()
        pltpu.make_async_copy(ei_hbm.at[1].at[pl.ds(om, K)],
                              idx_buf.at[slot, 1], sems_i[slot]).wait()

    def correct_idx(j, slot):
        o = (base + j) * K
        for q in range(K // L):
            gi = o + q * L + lax.iota(jnp.int32, L)
            real = gi < E
            lv = jnp.where(gi < E + N, gi - E, N)
            sv = idx_buf[slot, 0, pl.ds(q * L, L)]
            dv = idx_buf[slot, 1, pl.ds(q * L, L)]
            idx_buf[slot, 0, pl.ds(q * L, L)] = jnp.where(real, sv, lv)
            idx_buf[slot, 1, pl.ds(q * L, L)] = jnp.where(real, dv, lv)

    def wait_scatter(slot):
        pltpu.make_async_copy(row_buf.at[slot],
                              acc_sh.at[idx_buf.at[slot, 1]],
                              sems_s[slot]).wait()

    issue_idx(0, 0)
    pltpu.sync_copy(asrc_hbm, asrc_v)
    pltpu.sync_copy(adst_hbm, adst_v)

    zero16 = jnp.zeros((L,), jnp.float32)

    def zden(i, _):
        den_v[pl.ds(i * L, L)] = zero16
        return 0
    lax.fori_loop(0, N_PAD // L, zden, 0)

    def zrow(r, _):
        for q in range(C // L):
            row_buf[0, r, pl.ds(q * L, L)] = zero16
        return 0
    lax.fori_loop(0, K, zrow, 0)

    for b in range(NPT // K):
        pltpu.sync_copy(row_buf.at[0], acc_sh.at[pl.ds(s * NPT + b * K, K)])
    plsc.subcore_barrier()

    def issue_gather(j, rslot, islot):
        pltpu.async_copy(xl_hbm.at[idx_buf.at[islot, 0]], row_buf.at[rslot],
                         sems_r[rslot])

    def wait_gather(rslot, islot):
        pltpu.make_async_copy(xl_hbm.at[idx_buf.at[islot, 0]],
                              row_buf.at[rslot], sems_r[rslot]).wait()

    def step(j, u, first):
        slot, other = u % 2, 1 - (u % 2)
        islot, inext = u % 3, (u + 1) % 3
        for q in range(K // L):
            sv = idx_buf[islot, 0, pl.ds(q * L, L)]
            dv = idx_buf[islot, 1, pl.ds(q * L, L)]
            e = plsc.load_gather(asrc_v, [sv]) + plsc.load_gather(adst_v, [dv])
            e = jnp.where(e >= 0.0, e, e * 0.2)
            w = jnp.exp(e)
            plsc.addupdate_scatter(den_v, [dv], w)
            w_buf[pl.ds(q * L, L)] = w
        wait_idx(jnp.minimum(j + 1, nit - 1), inext)
        correct_idx(jnp.minimum(j + 1, nit - 1), inext)
        if not first:
            wait_scatter(other)
        issue_idx(jnp.minimum(j + 2, nit - 1), (u + 2) % 3)
        issue_gather(jnp.minimum(j + 1, nit - 1), other, inext)
        wait_gather(slot, islot)

        def scale(q16, _):
            w16 = w_buf[pl.ds(q16 * L, L)]
            for uu in range(L):
                r = q16 * L + uu
                wsplat = w16.at[jnp.full((L,), uu, jnp.int32)].get(
                    mode="promise_in_bounds")
                for q in range(C // L):
                    row_buf[slot, r, pl.ds(q * L, L)] = (
                        row_buf[slot, r, pl.ds(q * L, L)] * wsplat)
            return 0
        lax.fori_loop(0, K // L, scale, 0)
        pltpu.async_copy(row_buf.at[slot], acc_sh.at[idx_buf.at[islot, 1]],
                         sems_s[slot], add=True)

    issue_idx(jnp.int32(1), 1)
    wait_idx(jnp.int32(0), 0)
    correct_idx(jnp.int32(0), 0)
    issue_gather(jnp.int32(0), 0, 0)
    for u in range(6):
        step(jnp.int32(u), u, u == 0)

    def group(jj, _):
        for u in range(6):
            step(jj * 6 + u, u, False)
        return 0
    lax.fori_loop(1, ngrp, group, 0)

    wait_scatter(1)
    wait_gather(0, 0)
    wait_idx(nit - 1, 1)

    pltpu.sync_copy(den_v, den_out.at[g])

    plsc.subcore_barrier()
    pltpu.sync_copy(acc_sh.at[pl.ds(s * NPT, NPT)],
                    acc_out.at[c].at[pl.ds(s * NPT, NPT)])


def _sc_call(ei, xl, asrc, adst):
    mesh = plsc.VectorSubcoreMesh(core_axis_name="c", subcore_axis_name="s",
                                  num_cores=NC, num_subcores=NS)
    f = pl.kernel(
        _sc_body,
        out_type=[
            jax.ShapeDtypeStruct((NC, N_PAD, C), jnp.float32),
            jax.ShapeDtypeStruct((NW, N_PAD), jnp.float32),
        ],
        mesh=mesh,
        scratch_types=[
            pltpu.VMEM((N_PAD,), jnp.float32),
            pltpu.VMEM((N_PAD,), jnp.float32),
            pltpu.VMEM((N_PAD,), jnp.float32),
            pltpu.VMEM((3, 2, K), jnp.int32),
            pltpu.VMEM((2, K, C), jnp.float32),
            pltpu.VMEM((K,), jnp.float32),
            pltpu.VMEM_SHARED((N_PAD, C), jnp.float32),
            pltpu.SemaphoreType.DMA,
            pltpu.SemaphoreType.DMA,
            pltpu.SemaphoreType.DMA,
            pltpu.SemaphoreType.DMA,
            pltpu.SemaphoreType.DMA,
            pltpu.SemaphoreType.DMA,
            pltpu.SemaphoreType.DMA,
        ],
        compiler_params=pltpu.CompilerParams(needs_layout_passes=False),
    )
    return f(ei, xl, asrc, adst)


def _tc_post_body(acc0_ref, acc1_ref, den_ref, batch_ref,
                  bc_ref, wo_ref, bo_ref, y_ref, g_sc, cnt_sc):
    i = pl.program_id(0)

    @pl.when(i == 0)
    def _():
        g_sc[...] = jnp.zeros((G, C), jnp.float32)
        cnt_sc[...] = jnp.zeros((G, 1), jnp.float32)

    den_row = jnp.sum(den_ref[...], axis=0, keepdims=True) + 1e-16
    iden = (lax.broadcasted_iota(jnp.int32, (BR, BR), 0)
            == lax.broadcasted_iota(jnp.int32, (BR, BR), 1)).astype(jnp.float32)
    den_col = lax.dot_general(iden, den_row, (((1,), (1,)), ((), ())),
                              preferred_element_type=jnp.float32)
    h2 = (acc0_ref[...] + acc1_ref[...]) / den_col + bc_ref[...]
    h2 = jnp.maximum(h2, 0.0)
    b = batch_ref[0]
    oh = (b == lax.broadcasted_iota(jnp.int32, (BR, G), 1)).astype(jnp.float32)
    g_sc[...] += lax.dot_general(oh, h2, (((0,), (0,)), ((), ())),
                                 preferred_element_type=jnp.float32)
    ones = jnp.ones((BR, 1), jnp.float32)
    cnt_sc[...] += lax.dot_general(oh, ones, (((0,), (0,)), ((), ())),
                                   preferred_element_type=jnp.float32)

    @pl.when(i == NBLK - 1)
    def _():
        gm = g_sc[...] / jnp.maximum(cnt_sc[...], 1.0)
        y_ref[...] = jnp.dot(gm, wo_ref[...],
                             preferred_element_type=jnp.float32) + bo_ref[...]


def _tc_post(acc0, acc1, den4, batch3, b_conv, W_out, b_out):
    return pl.pallas_call(
        _tc_post_body,
        grid=(NBLK,),
        in_specs=[
            pl.BlockSpec((BR, C), lambda i: (i, 0)),
            pl.BlockSpec((BR, C), lambda i: (i, 0)),
            pl.BlockSpec((NW, BR), lambda i: (0, i)),
            pl.BlockSpec((1, BR, 1), lambda i: (i, 0, 0)),
            pl.BlockSpec((1, C), lambda i: (0, 0)),
            pl.BlockSpec((C, 1), lambda i: (0, 0)),
            pl.BlockSpec((1, 1), lambda i: (0, 0)),
        ],
        out_specs=pl.BlockSpec((G, 1), lambda i: (0, 0)),
        out_shape=jax.ShapeDtypeStruct((G, 1), jnp.float32),
        scratch_shapes=[
            pltpu.VMEM((G, C), jnp.float32),
            pltpu.VMEM((G, 1), jnp.float32),
        ],
    )(acc0, acc1, den4, batch3, b_conv, W_out, b_out)


def kernel(x, edge_index, batch, W_in, b_in, W_conv, att_src, att_dst,
           b_conv, W_out, b_out):
    x_pad = jnp.zeros((N_PAD, D), jnp.float32).at[:N].set(x)
    att_s = att_src.reshape(1, C)
    att_d = att_dst.reshape(1, C)

    xl, asrc, adst = _tc_pre(x_pad, W_in, b_in.reshape(1, C), W_conv,
                             att_s, att_d)

    acc, den = _sc_call(edge_index, xl, asrc.reshape(N_PAD),
                        adst.reshape(N_PAD))

    batch3 = jnp.concatenate(
        [batch, jnp.full((N_PAD - N,), G, jnp.int32)]).reshape(NBLK, BR, 1)
    y = _tc_post(acc[0], acc[1], den,
                 batch3, b_conv.reshape(1, C), W_out, b_out.reshape(1, 1))
    return y


# speedup vs baseline: 1.1650x; 1.1650x over previous
import functools

import jax
import jax.numpy as jnp
from jax import lax
from jax.experimental import pallas as pl
from jax.experimental.pallas import tpu as pltpu
from jax.experimental.pallas import tpu_sc as plsc

N = 10000
E = 320000
D = 128
C = 128
G = 64

NC, NS, L = 2, 16, 16
NW = NC * NS
N_PAD = 10240
NPT = N_PAD // NS
K = 64
NIT0 = 192
NIT1 = 132
NITP = NIT0 + NIT1
E_PAD = NS * NITP * K
BR = 512
NBLK = N_PAD // BR


def _tc_pre_body(x_ref, wi_ref, bi_ref, wc_ref, as_ref, ad_ref,
                 xl_ref, asrc_ref, adst_ref):
    h = jnp.dot(x_ref[...], wi_ref[...], preferred_element_type=jnp.float32)
    h = h + bi_ref[...]
    xl = jnp.dot(h, wc_ref[...], preferred_element_type=jnp.float32)
    xl_ref[...] = xl
    asrc_ref[...] = jnp.sum(xl * as_ref[...], axis=1, keepdims=True)
    adst_ref[...] = jnp.sum(xl * ad_ref[...], axis=1, keepdims=True)


def _tc_pre(x_pad, W_in, b_in, W_conv, att_s, att_d):
    return pl.pallas_call(
        _tc_pre_body,
        grid=(NBLK,),
        in_specs=[
            pl.BlockSpec((BR, D), lambda i: (i, 0)),
            pl.BlockSpec((D, C), lambda i: (0, 0)),
            pl.BlockSpec((1, C), lambda i: (0, 0)),
            pl.BlockSpec((C, C), lambda i: (0, 0)),
            pl.BlockSpec((1, C), lambda i: (0, 0)),
            pl.BlockSpec((1, C), lambda i: (0, 0)),
        ],
        out_specs=[
            pl.BlockSpec((BR, C), lambda i: (i, 0)),
            pl.BlockSpec((BR, 1), lambda i: (i, 0)),
            pl.BlockSpec((BR, 1), lambda i: (i, 0)),
        ],
        out_shape=[
            jax.ShapeDtypeStruct((N_PAD, C), jnp.float32),
            jax.ShapeDtypeStruct((N_PAD, 1), jnp.float32),
            jax.ShapeDtypeStruct((N_PAD, 1), jnp.float32),
        ],
    )(x_pad, W_in, b_in, W_conv, att_s, att_d)


def _sc_body(ei_hbm, xl_hbm, asrc_hbm, adst_hbm,
             acc_out, den_out,
             asrc_v, adst_v, den_v, idx_buf, row_buf, w_buf,
             acc_sh, sem_i0, sem_i1, sem_i2, sem_r0, sem_r1, sem_s0, sem_s1):
    c = lax.axis_index("c")
    s = lax.axis_index("s")
    g = c * NS + s
    sems_i = (sem_i0, sem_i1, sem_i2)
    sems_r = (sem_r0, sem_r1)
    sems_s = (sem_s0, sem_s1)

    base = s * NITP + jnp.where(c == 0, 0, NIT0)
    nit = jnp.where(c == 0, NIT0, NIT1)
    ngrp = jnp.where(c == 0, NIT0 // 6, NIT1 // 6)

    def issue_idx(j, slot):
        om = jnp.minimum((base + j) * K, E - K)
        pltpu.async_copy(ei_hbm.at[0].at[pl.ds(om, K)], idx_buf.at[slot, 0],
                         sems_i[slot])
        pltpu.async_copy(ei_hbm.at[1].at[pl.ds(om, K)], idx_buf.at[slot, 1],
                         sems_i[slot])

    def wait_idx(j, slot):
        om = jnp.minimum((base + j) * K, E - K)
        pltpu.make_async_copy(ei_hbm.at[0].at[pl.ds(om, K)],
                              idx_buf.at[slot, 0], sems_i[slot]).wait()
        pltpu.make_async_copy(ei_hbm.at[1].at[pl.ds(om, K)],
                              idx_buf.at[slot, 1], sems_i[slot]).wait()

    def correct_idx(j, slot):
        o = (base + j) * K
        for q in range(K // L):
            gi = o + q * L + lax.iota(jnp.int32, L)
            real = gi < E
            lv = jnp.where(gi < E + N, gi - E, N)
            sv = idx_buf[slot, 0, pl.ds(q * L, L)]
            dv = idx_buf[slot, 1, pl.ds(q * L, L)]
            idx_buf[slot, 0, pl.ds(q * L, L)] = jnp.where(real, sv, lv)
            idx_buf[slot, 1, pl.ds(q * L, L)] = jnp.where(real, dv, lv)

    def wait_scatter(slot):
        pltpu.make_async_copy(row_buf.at[slot],
                              acc_sh.at[idx_buf.at[slot, 1]],
                              sems_s[slot]).wait()

    issue_idx(0, 0)
    pltpu.sync_copy(asrc_hbm, asrc_v)
    pltpu.sync_copy(adst_hbm, adst_v)

    zero16 = jnp.zeros((L,), jnp.float32)

    def zden(i, _):
        den_v[pl.ds(i * L, L)] = zero16
        return 0
    lax.fori_loop(0, N_PAD // L, zden, 0)

    def zrow(r, _):
        for q in range(C // L):
            row_buf[0, r, pl.ds(q * L, L)] = zero16
        return 0
    lax.fori_loop(0, K, zrow, 0)

    for b in range(NPT // K):
        pltpu.sync_copy(row_buf.at[0], acc_sh.at[pl.ds(s * NPT + b * K, K)])
    plsc.subcore_barrier()

    def issue_gather(j, rslot, islot):
        pltpu.async_copy(xl_hbm.at[idx_buf.at[islot, 0]], row_buf.at[rslot],
                         sems_r[rslot])

    def wait_gather(rslot, islot):
        pltpu.make_async_copy(xl_hbm.at[idx_buf.at[islot, 0]],
                              row_buf.at[rslot], sems_r[rslot]).wait()

    def step(j, u, first):
        slot, other = u % 2, 1 - (u % 2)
        islot, inext = u % 3, (u + 1) % 3
        for q in range(K // L):
            sv = idx_buf[islot, 0, pl.ds(q * L, L)]
            dv = idx_buf[islot, 1, pl.ds(q * L, L)]
            e = plsc.load_gather(asrc_v, [sv]) + plsc.load_gather(adst_v, [dv])
            e = jnp.where(e >= 0.0, e, e * 0.2)
            w = jnp.exp(e)
            plsc.addupdate_scatter(den_v, [dv], w)
            w_buf[pl.ds(q * L, L)] = w
        wait_idx(jnp.minimum(j + 1, nit - 1), inext)
        correct_idx(jnp.minimum(j + 1, nit - 1), inext)
        if not first:
            wait_scatter(other)
        issue_idx(jnp.minimum(j + 2, nit - 1), (u + 2) % 3)
        issue_gather(jnp.minimum(j + 1, nit - 1), other, inext)
        wait_gather(slot, islot)

        def scale(q16, _):
            w16 = w_buf[pl.ds(q16 * L, L)]
            for uu in range(L):
                r = q16 * L + uu
                wsplat = w16.at[jnp.full((L,), uu, jnp.int32)].get(
                    mode="promise_in_bounds")
                for q in range(C // L):
                    row_buf[slot, r, pl.ds(q * L, L)] = (
                        row_buf[slot, r, pl.ds(q * L, L)] * wsplat)
            return 0
        lax.fori_loop(0, K // L, scale, 0)
        pltpu.async_copy(row_buf.at[slot], acc_sh.at[idx_buf.at[islot, 1]],
                         sems_s[slot], add=True)

    issue_idx(jnp.int32(1), 1)
    wait_idx(jnp.int32(0), 0)
    correct_idx(jnp.int32(0), 0)
    issue_gather(jnp.int32(0), 0, 0)
    for u in range(6):
        step(jnp.int32(u), u, u == 0)

    def group(jj, _):
        for u in range(6):
            step(jj * 6 + u, u, False)
        return 0
    lax.fori_loop(1, ngrp, group, 0)

    wait_scatter(1)
    wait_gather(0, 0)
    wait_idx(nit - 1, 1)

    pltpu.sync_copy(den_v, den_out.at[g])

    plsc.subcore_barrier()
    pltpu.sync_copy(acc_sh.at[pl.ds(s * NPT, NPT)],
                    acc_out.at[c].at[pl.ds(s * NPT, NPT)])


def _sc_call(ei, xl, asrc, adst):
    mesh = plsc.VectorSubcoreMesh(core_axis_name="c", subcore_axis_name="s",
                                  num_cores=NC, num_subcores=NS)
    f = pl.kernel(
        _sc_body,
        out_type=[
            jax.ShapeDtypeStruct((NC, N_PAD, C), jnp.float32),
            jax.ShapeDtypeStruct((NW, N_PAD), jnp.float32),
        ],
        mesh=mesh,
        scratch_types=[
            pltpu.VMEM((N_PAD,), jnp.float32),
            pltpu.VMEM((N_PAD,), jnp.float32),
            pltpu.VMEM((N_PAD,), jnp.float32),
            pltpu.VMEM((3, 2, K), jnp.int32),
            pltpu.VMEM((2, K, C), jnp.float32),
            pltpu.VMEM((K,), jnp.float32),
            pltpu.VMEM_SHARED((N_PAD, C), jnp.float32),
            pltpu.SemaphoreType.DMA,
            pltpu.SemaphoreType.DMA,
            pltpu.SemaphoreType.DMA,
            pltpu.SemaphoreType.DMA,
            pltpu.SemaphoreType.DMA,
            pltpu.SemaphoreType.DMA,
            pltpu.SemaphoreType.DMA,
        ],
        compiler_params=pltpu.CompilerParams(needs_layout_passes=False),
    )
    return f(ei, xl, asrc, adst)


def _tc_post_body(acc0_ref, acc1_ref, den_ref, batch_ref,
                  bc_ref, wo_ref, bo_ref, y_ref, g_sc, cnt_sc):
    i = pl.program_id(0)

    @pl.when(i == 0)
    def _():
        g_sc[...] = jnp.zeros((G, C), jnp.float32)
        cnt_sc[...] = jnp.zeros((G, 1), jnp.float32)

    den_row = jnp.sum(den_ref[...], axis=0, keepdims=True) + 1e-16
    iden = (lax.broadcasted_iota(jnp.int32, (BR, BR), 0)
            == lax.broadcasted_iota(jnp.int32, (BR, BR), 1)).astype(jnp.float32)
    den_col = lax.dot_general(iden, den_row, (((1,), (1,)), ((), ())),
                              preferred_element_type=jnp.float32)
    h2 = (acc0_ref[...] + acc1_ref[...]) / den_col + bc_ref[...]
    h2 = jnp.maximum(h2, 0.0)
    b = batch_ref[0]
    oh = (b == lax.broadcasted_iota(jnp.int32, (BR, G), 1)).astype(jnp.float32)
    g_sc[...] += lax.dot_general(oh, h2, (((0,), (0,)), ((), ())),
                                 preferred_element_type=jnp.float32)
    ones = jnp.ones((BR, 1), jnp.float32)
    cnt_sc[...] += lax.dot_general(oh, ones, (((0,), (0,)), ((), ())),
                                   preferred_element_type=jnp.float32)

    @pl.when(i == NBLK - 1)
    def _():
        gm = g_sc[...] / jnp.maximum(cnt_sc[...], 1.0)
        y_ref[...] = jnp.dot(gm, wo_ref[...],
                             preferred_element_type=jnp.float32) + bo_ref[...]


def _tc_post(acc0, acc1, den4, batch3, b_conv, W_out, b_out):
    return pl.pallas_call(
        _tc_post_body,
        grid=(NBLK,),
        in_specs=[
            pl.BlockSpec((BR, C), lambda i: (i, 0)),
            pl.BlockSpec((BR, C), lambda i: (i, 0)),
            pl.BlockSpec((NW, BR), lambda i: (0, i)),
            pl.BlockSpec((1, BR, 1), lambda i: (i, 0, 0)),
            pl.BlockSpec((1, C), lambda i: (0, 0)),
            pl.BlockSpec((C, 1), lambda i: (0, 0)),
            pl.BlockSpec((1, 1), lambda i: (0, 0)),
        ],
        out_specs=pl.BlockSpec((G, 1), lambda i: (0, 0)),
        out_shape=jax.ShapeDtypeStruct((G, 1), jnp.float32),
        scratch_shapes=[
            pltpu.VMEM((G, C), jnp.float32),
            pltpu.VMEM((G, 1), jnp.float32),
        ],
    )(acc0, acc1, den4, batch3, b_conv, W_out, b_out)


def kernel(x, edge_index, batch, W_in, b_in, W_conv, att_src, att_dst,
           b_conv, W_out, b_out):
    x_pad = jnp.zeros((N_PAD, D), jnp.float32).at[:N].set(x)
    att_s = att_src.reshape(1, C)
    att_d = att_dst.reshape(1, C)

    xl, asrc, adst = _tc_pre(x_pad, W_in, b_in.reshape(1, C), W_conv,
                             att_s, att_d)

    acc, den = _sc_call(edge_index, xl, asrc.reshape(N_PAD),
                        adst.reshape(N_PAD))

    batch3 = jnp.concatenate(
        [batch, jnp.full((N_PAD - N,), G, jnp.int32)]).reshape(NBLK, BR, 1)
    y = _tc_post(acc[0], acc[1], den,
                 batch3, b_conv.reshape(1, C), W_out, b_out.reshape(1, 1))
    return y


# speedup vs baseline: 1.1662x; 1.0011x over previous
import jax
import jax.numpy as jnp
from jax import lax
from jax.experimental import pallas as pl
from jax.experimental.pallas import tpu as pltpu
from jax.experimental.pallas import tpu_sc as plsc

N = 10000
E = 320000
D = 128
C = 128
G = 64

NC, NS, L = 2, 16, 16
NW = NC * NS
N_PAD = 10240
NPT = N_PAD // NS
K = 64
NIT0 = 192
NIT1 = 132
NITP = NIT0 + NIT1
E_PAD = NS * NITP * K
BR = 512
NBLK = N_PAD // BR


def _tc_pre_body(x_ref, wi_ref, bi_ref, wc_ref, as_ref, ad_ref,
                 xl_ref, asrc_ref, adst_ref):
    h = jnp.dot(x_ref[...], wi_ref[...], preferred_element_type=jnp.float32)
    h = h + bi_ref[...]
    xl = jnp.dot(h, wc_ref[...], preferred_element_type=jnp.float32)
    xl_ref[...] = xl
    asrc_ref[...] = jnp.sum(xl * as_ref[...], axis=1, keepdims=True)
    adst_ref[...] = jnp.sum(xl * ad_ref[...], axis=1, keepdims=True)


def _tc_pre(x_pad, W_in, b_in, W_conv, att_s, att_d):
    return pl.pallas_call(
        _tc_pre_body,
        grid=(NBLK,),
        in_specs=[
            pl.BlockSpec((BR, D), lambda i: (i, 0)),
            pl.BlockSpec((D, C), lambda i: (0, 0)),
            pl.BlockSpec((1, C), lambda i: (0, 0)),
            pl.BlockSpec((C, C), lambda i: (0, 0)),
            pl.BlockSpec((1, C), lambda i: (0, 0)),
            pl.BlockSpec((1, C), lambda i: (0, 0)),
        ],
        out_specs=[
            pl.BlockSpec((BR, C), lambda i: (i, 0)),
            pl.BlockSpec((BR, 1), lambda i: (i, 0)),
            pl.BlockSpec((BR, 1), lambda i: (i, 0)),
        ],
        out_shape=[
            jax.ShapeDtypeStruct((N_PAD, C), jnp.float32),
            jax.ShapeDtypeStruct((N_PAD, 1), jnp.float32),
            jax.ShapeDtypeStruct((N_PAD, 1), jnp.float32),
        ],
    )(x_pad, W_in, b_in, W_conv, att_s, att_d)


def _sc_body(ei_hbm, xl_hbm, asrc_hbm, adst_hbm,
             acc_out, den_out,
             asrc_v, adst_v, den_v, idx_buf, row_buf, w_buf,
             acc_sh, sem_i0, sem_i1, sem_i2, sem_r0, sem_r1, sem_s0, sem_s1):
    c = lax.axis_index("c")
    s = lax.axis_index("s")
    g = c * NS + s
    sems_i = (sem_i0, sem_i1, sem_i2)
    sems_r = (sem_r0, sem_r1)
    sems_s = (sem_s0, sem_s1)

    base = s * NITP + jnp.where(c == 0, 0, NIT0)
    nit = jnp.where(c == 0, NIT0, NIT1)
    ngrp = jnp.where(c == 0, NIT0 // 6, NIT1 // 6)

    def issue_idx(j, slot):
        om = jnp.minimum((base + j) * K, E - K)
        pltpu.async_copy(ei_hbm.at[0].at[pl.ds(om, K)], idx_buf.at[slot, 0],
                         sems_i[slot])
        pltpu.async_copy(ei_hbm.at[1].at[pl.ds(om, K)], idx_buf.at[slot, 1],
                         sems_i[slot])

    def wait_idx(j, slot):
        om = jnp.minimum((base + j) * K, E - K)
        pltpu.make_async_copy(ei_hbm.at[0].at[pl.ds(om, K)],
                              idx_buf.at[slot, 0], sems_i[slot]).wait()
        pltpu.make_async_copy(ei_hbm.at[1].at[pl.ds(om, K)],
                              idx_buf.at[slot, 1], sems_i[slot]).wait()

    def correct_idx(j, slot):
        o = (base + j) * K
        for q in range(K // L):
            gi = o + q * L + lax.iota(jnp.int32, L)
            real = gi < E
            lv = jnp.where(gi < E + N, gi - E, N)
            sv = idx_buf[slot, 0, pl.ds(q * L, L)]
            dv = idx_buf[slot, 1, pl.ds(q * L, L)]
            idx_buf[slot, 0, pl.ds(q * L, L)] = jnp.where(real, sv, lv)
            idx_buf[slot, 1, pl.ds(q * L, L)] = jnp.where(real, dv, lv)

    def wait_scatter(slot):
        pltpu.make_async_copy(row_buf.at[slot],
                              acc_sh.at[idx_buf.at[slot, 1]],
                              sems_s[slot]).wait()

    issue_idx(0, 0)
    pltpu.sync_copy(asrc_hbm, asrc_v)
    pltpu.sync_copy(adst_hbm, adst_v)

    zero16 = jnp.zeros((L,), jnp.float32)

    def zden(i, _):
        den_v[pl.ds(i * L, L)] = zero16
        return 0
    lax.fori_loop(0, N_PAD // L, zden, 0)

    def zrow(r, _):
        for q in range(C // L):
            row_buf[0, r, pl.ds(q * L, L)] = zero16
        return 0
    lax.fori_loop(0, K, zrow, 0)

    for b in range(NPT // K):
        pltpu.sync_copy(row_buf.at[0], acc_sh.at[pl.ds(s * NPT + b * K, K)])
    plsc.subcore_barrier()

    def issue_gather(j, rslot, islot):
        pltpu.async_copy(xl_hbm.at[idx_buf.at[islot, 0]], row_buf.at[rslot],
                         sems_r[rslot])

    def wait_gather(rslot, islot):
        pltpu.make_async_copy(xl_hbm.at[idx_buf.at[islot, 0]],
                              row_buf.at[rslot], sems_r[rslot]).wait()

    def step(j, u, first):
        slot, other = u % 2, 1 - (u % 2)
        islot, inext = u % 3, (u + 1) % 3
        for q in range(K // L):
            sv = idx_buf[islot, 0, pl.ds(q * L, L)]
            dv = idx_buf[islot, 1, pl.ds(q * L, L)]
            e = plsc.load_gather(asrc_v, [sv]) + plsc.load_gather(adst_v, [dv])
            e = jnp.where(e >= 0.0, e, e * 0.2)
            w = jnp.exp(e)
            plsc.addupdate_scatter(den_v, [dv], w)
            w_buf[pl.ds(q * L, L)] = w
        wait_idx(jnp.minimum(j + 1, nit - 1), inext)
        correct_idx(jnp.minimum(j + 1, nit - 1), inext)
        if not first:
            wait_scatter(other)
        issue_idx(jnp.minimum(j + 2, nit - 1), (u + 2) % 3)
        issue_gather(jnp.minimum(j + 1, nit - 1), other, inext)
        wait_gather(slot, islot)

        def scale(q16, _):
            w16 = w_buf[pl.ds(q16 * L, L)]
            for uu in range(L):
                r = q16 * L + uu
                wsplat = w16.at[jnp.full((L,), uu, jnp.int32)].get(
                    mode="promise_in_bounds")
                for q in range(C // L):
                    row_buf[slot, r, pl.ds(q * L, L)] = (
                        row_buf[slot, r, pl.ds(q * L, L)] * wsplat)
            return 0
        lax.fori_loop(0, K // L, scale, 0)
        pltpu.async_copy(row_buf.at[slot], acc_sh.at[idx_buf.at[islot, 1]],
                         sems_s[slot], add=True)

    issue_idx(jnp.int32(1), 1)
    wait_idx(jnp.int32(0), 0)
    correct_idx(jnp.int32(0), 0)
    issue_gather(jnp.int32(0), 0, 0)
    for u in range(6):
        step(jnp.int32(u), u, u == 0)

    def group(jj, _):
        for u in range(6):
            step(jj * 6 + u, u, False)
        return 0
    lax.fori_loop(1, ngrp, group, 0)

    wait_scatter(1)
    wait_gather(0, 0)
    wait_idx(nit - 1, 1)

    pltpu.sync_copy(den_v, den_out.at[g])

    plsc.subcore_barrier()
    pltpu.sync_copy(acc_sh.at[pl.ds(s * NPT, NPT)],
                    acc_out.at[c].at[pl.ds(s * NPT, NPT)])


def _sc_call(ei, xl, asrc, adst):
    mesh = plsc.VectorSubcoreMesh(core_axis_name="c", subcore_axis_name="s",
                                  num_cores=NC, num_subcores=NS)
    f = pl.kernel(
        _sc_body,
        out_type=[
            jax.ShapeDtypeStruct((NC, N_PAD, C), jnp.float32),
            jax.ShapeDtypeStruct((NW, N_PAD), jnp.float32),
        ],
        mesh=mesh,
        scratch_types=[
            pltpu.VMEM((N_PAD,), jnp.float32),
            pltpu.VMEM((N_PAD,), jnp.float32),
            pltpu.VMEM((N_PAD,), jnp.float32),
            pltpu.VMEM((3, 2, K), jnp.int32),
            pltpu.VMEM((2, K, C), jnp.float32),
            pltpu.VMEM((K,), jnp.float32),
            pltpu.VMEM_SHARED((N_PAD, C), jnp.float32),
            pltpu.SemaphoreType.DMA,
            pltpu.SemaphoreType.DMA,
            pltpu.SemaphoreType.DMA,
            pltpu.SemaphoreType.DMA,
            pltpu.SemaphoreType.DMA,
            pltpu.SemaphoreType.DMA,
            pltpu.SemaphoreType.DMA,
        ],
        compiler_params=pltpu.CompilerParams(needs_layout_passes=False),
    )
    return f(ei, xl, asrc, adst)


def _tc_post_body(acc0_ref, acc1_ref, den_ref, batch_ref,
                  bc_ref, wo_ref, bo_ref, y_ref, g_sc, cnt_sc):
    i = pl.program_id(0)

    @pl.when(i == 0)
    def _():
        g_sc[...] = jnp.zeros((G, C), jnp.float32)
        cnt_sc[...] = jnp.zeros((G, 1), jnp.float32)

    den_row = jnp.sum(den_ref[...], axis=0, keepdims=True) + 1e-16
    iden = (lax.broadcasted_iota(jnp.int32, (BR, BR), 0)
            == lax.broadcasted_iota(jnp.int32, (BR, BR), 1)).astype(jnp.float32)
    den_col = lax.dot_general(iden, den_row, (((1,), (1,)), ((), ())),
                              preferred_element_type=jnp.float32)
    h2 = (acc0_ref[...] + acc1_ref[...]) / den_col + bc_ref[...]
    h2 = jnp.maximum(h2, 0.0)
    b = batch_ref[0]
    oh = (b == lax.broadcasted_iota(jnp.int32, (BR, G), 1)).astype(jnp.float32)
    g_sc[...] += lax.dot_general(oh, h2, (((0,), (0,)), ((), ())),
                                 preferred_element_type=jnp.float32)
    ones = jnp.ones((BR, 1), jnp.float32)
    cnt_sc[...] += lax.dot_general(oh, ones, (((0,), (0,)), ((), ())),
                                   preferred_element_type=jnp.float32)

    @pl.when(i == NBLK - 1)
    def _():
        gm = g_sc[...] / jnp.maximum(cnt_sc[...], 1.0)
        y_ref[...] = jnp.dot(gm, wo_ref[...],
                             preferred_element_type=jnp.float32) + bo_ref[...]


def _tc_post(acc0, acc1, den4, batch3, b_conv, W_out, b_out):
    return pl.pallas_call(
        _tc_post_body,
        grid=(NBLK,),
        in_specs=[
            pl.BlockSpec((BR, C), lambda i: (i, 0)),
            pl.BlockSpec((BR, C), lambda i: (i, 0)),
            pl.BlockSpec((NW, BR), lambda i: (0, i)),
            pl.BlockSpec((1, BR, 1), lambda i: (i, 0, 0)),
            pl.BlockSpec((1, C), lambda i: (0, 0)),
            pl.BlockSpec((C, 1), lambda i: (0, 0)),
            pl.BlockSpec((1, 1), lambda i: (0, 0)),
        ],
        out_specs=pl.BlockSpec((G, 1), lambda i: (0, 0)),
        out_shape=jax.ShapeDtypeStruct((G, 1), jnp.float32),
        scratch_shapes=[
            pltpu.VMEM((G, C), jnp.float32),
            pltpu.VMEM((G, 1), jnp.float32),
        ],
    )(acc0, acc1, den4, batch3, b_conv, W_out, b_out)


def kernel(x, edge_index, batch, W_in, b_in, W_conv, att_src, att_dst,
           b_conv, W_out, b_out):
    x_pad = jnp.zeros((N_PAD, D), jnp.float32).at[:N].set(x)
    att_s = att_src.reshape(1, C)
    att_d = att_dst.reshape(1, C)

    xl, asrc, adst = _tc_pre(x_pad, W_in, b_in.reshape(1, C), W_conv,
                             att_s, att_d)

    acc, den = _sc_call(edge_index, xl, asrc.reshape(N_PAD),
                        adst.reshape(N_PAD))

    batch3 = jnp.concatenate(
        [batch, jnp.full((N_PAD - N,), G, jnp.int32)]).reshape(NBLK, BR, 1)
    y = _tc_post(acc[0], acc[1], den,
                 batch3, b_conv.reshape(1, C), W_out, b_out.reshape(1, 1))
    return y
